# baseline (speedup 1.0000x reference)
import functools

import jax
import jax.numpy as jnp
from jax.experimental import pallas as pl
from jax.experimental.pallas import tpu as pltpu

B, S, D_MODEL = 4, 2048, 4096
ROUTER = 1024
NUM_EXPERTS = 64
TOP_K = 8
THRESHOLD = 0.1
LATERAL = 0.1

CHUNK = 256
NSTEPS = S // CHUNK


def _fused_body(hid_ref, w1_ref, b1_ref, g_ref, be_ref, w2_ref, b2_ref,
                rs_ref, feat_ref, sel_ref, wts_ref, acc_ref):
    i = pl.program_id(0)

    @pl.when(i == 0)
    def _init():
        acc_ref[...] = jnp.zeros_like(acc_ref)

    acc_ref[...] += jnp.sum(hid_ref[...], axis=1)

    @pl.when(i == NSTEPS - 1)
    def _head():
        pooled = acc_ref[...] * (1.0 / S)
        h = jax.lax.dot_general(
            pooled, w1_ref[...], (((1,), (1,)), ((), ())),
            preferred_element_type=jnp.float32) + b1_ref[...]
        mu = jnp.mean(h, axis=-1, keepdims=True)
        var = jnp.mean((h - mu) ** 2, axis=-1, keepdims=True)
        h = (h - mu) / jnp.sqrt(var + 1e-5) * g_ref[...] + be_ref[...]
        h = 0.5 * h * (1.0 + jax.lax.erf(h * (2.0 ** -0.5)))
        features = jax.lax.dot_general(
            h, w2_ref[...], (((1,), (1,)), ((), ())),
            preferred_element_type=jnp.float32) + b2_ref[...]
        feat_ref[...] = features

        affinity = jax.lax.dot_general(
            features, w2_ref[0:NUM_EXPERTS, :], (((1,), (1,)), ((), ())),
            preferred_element_type=jnp.float32)
        logits = rs_ref[...] + 0.1 * affinity

        acts = jnp.maximum(logits - THRESHOLD, 0.0)
        for _ in range(3):
            total = jnp.sum(acts, axis=-1, keepdims=True)
            inhibition = LATERAL * (total - acts)
            acts = jnp.maximum(logits - THRESHOLD - inhibition, 0.0)

        idx = jax.lax.broadcasted_iota(jnp.int32, (B, NUM_EXPERTS), 1)
        kidx = jax.lax.broadcasted_iota(jnp.int32, (B, TOP_K), 1)
        work = acts
        vals = jnp.zeros((B, TOP_K), jnp.float32)
        sel = jnp.zeros((B, TOP_K), jnp.int32)
        for k in range(TOP_K):
            m = jnp.max(work, axis=-1, keepdims=True)
            first = jnp.min(jnp.where(work == m, idx, NUM_EXPERTS),
                            axis=-1, keepdims=True)
            vals = jnp.where(kidx == k, m, vals)
            sel = jnp.where(kidx == k, first, sel)
            work = jnp.where(idx == first, -jnp.inf, work)
        wmax = jnp.max(vals, axis=-1, keepdims=True)
        ex = jnp.exp(vals - wmax)
        wts_ref[...] = ex / jnp.sum(ex, axis=-1, keepdims=True)
        sel_ref[...] = sel


@jax.jit
def _run(hidden_states, W1, b1, gamma, beta, W2, b2, routing_scores):
    features, sel, wts = pl.pallas_call(
        _fused_body,
        grid=(NSTEPS,),
        in_specs=[
            pl.BlockSpec((B, CHUNK, D_MODEL), lambda i: (0, i, 0)),
            pl.BlockSpec((ROUTER, D_MODEL), lambda i: (0, 0)),
            pl.BlockSpec((1, ROUTER), lambda i: (0, 0)),
            pl.BlockSpec((1, ROUTER), lambda i: (0, 0)),
            pl.BlockSpec((1, ROUTER), lambda i: (0, 0)),
            pl.BlockSpec((ROUTER, ROUTER), lambda i: (0, 0)),
            pl.BlockSpec((1, ROUTER), lambda i: (0, 0)),
            pl.BlockSpec((1, NUM_EXPERTS), lambda i: (0, 0)),
        ],
        out_specs=[
            pl.BlockSpec((B, ROUTER), lambda i: (0, 0)),
            pl.BlockSpec((B, TOP_K), lambda i: (0, 0)),
            pl.BlockSpec((B, TOP_K), lambda i: (0, 0)),
        ],
        out_shape=[
            jax.ShapeDtypeStruct((B, ROUTER), jnp.float32),
            jax.ShapeDtypeStruct((B, TOP_K), jnp.int32),
            jax.ShapeDtypeStruct((B, TOP_K), jnp.float32),
        ],
        scratch_shapes=[pltpu.VMEM((B, D_MODEL), jnp.float32)],
        compiler_params=pltpu.CompilerParams(
            dimension_semantics=("arbitrary",)),
    )(hidden_states, W1, b1.reshape(1, -1), gamma.reshape(1, -1),
      beta.reshape(1, -1), W2, b2.reshape(1, -1),
      routing_scores.reshape(1, -1))
    return features, sel[0], wts


def kernel(hidden_states, W1, b1, gamma, beta, W2, b2, routing_scores):
    return _run(hidden_states, W1, b1, gamma, beta, W2, b2, routing_scores)


# speedup vs baseline: 1.1295x; 1.1295x over previous
import jax
import jax.numpy as jnp
from jax.experimental import pallas as pl
from jax.experimental.pallas import tpu as pltpu

B, S, D_MODEL = 4, 2048, 4096
ROUTER = 1024
NUM_EXPERTS = 64
TOP_K = 8
THRESHOLD = 0.1
LATERAL = 0.1

DCHUNK = 512
NSTEPS = D_MODEL // DCHUNK


def _fused_body(hid_ref, w1_ref, b1_ref, g_ref, be_ref, w2_ref, b2_ref,
                rs_ref, feat_ref, sel_ref, wts_ref, h_ref):
    i = pl.program_id(0)

    pooled_j = jnp.sum(hid_ref[...], axis=1) * (1.0 / S)
    hj = jax.lax.dot_general(
        pooled_j, w1_ref[...], (((1,), (1,)), ((), ())),
        preferred_element_type=jnp.float32)

    @pl.when(i == 0)
    def _init():
        h_ref[...] = hj

    @pl.when(i > 0)
    def _acc():
        h_ref[...] += hj

    @pl.when(i == NSTEPS - 1)
    def _head():
        h = h_ref[...] + b1_ref[...]
        mu = jnp.mean(h, axis=-1, keepdims=True)
        var = jnp.mean((h - mu) ** 2, axis=-1, keepdims=True)
        h = (h - mu) / jnp.sqrt(var + 1e-5) * g_ref[...] + be_ref[...]
        h = 0.5 * h * (1.0 + jax.lax.erf(h * (2.0 ** -0.5)))
        features = jax.lax.dot_general(
            h, w2_ref[...], (((1,), (1,)), ((), ())),
            preferred_element_type=jnp.float32) + b2_ref[...]
        feat_ref[...] = features

        affinity = jax.lax.dot_general(
            features, w2_ref[0:NUM_EXPERTS, :], (((1,), (1,)), ((), ())),
            preferred_element_type=jnp.float32)
        logits = rs_ref[...] + 0.1 * affinity

        acts = jnp.maximum(logits - THRESHOLD, 0.0)
        for _ in range(3):
            total = jnp.sum(acts, axis=-1, keepdims=True)
            inhibition = LATERAL * (total - acts)
            acts = jnp.maximum(logits - THRESHOLD - inhibition, 0.0)

        idx = jax.lax.broadcasted_iota(jnp.int32, (B, NUM_EXPERTS), 1)
        kidx = jax.lax.broadcasted_iota(jnp.int32, (B, TOP_K), 1)
        work = acts
        vals = jnp.zeros((B, TOP_K), jnp.float32)
        sel = jnp.zeros((B, TOP_K), jnp.int32)
        for k in range(TOP_K):
            m = jnp.max(work, axis=-1, keepdims=True)
            first = jnp.min(jnp.where(work == m, idx, NUM_EXPERTS),
                            axis=-1, keepdims=True)
            vals = jnp.where(kidx == k, m, vals)
            sel = jnp.where(kidx == k, first, sel)
            work = jnp.where(idx == first, -jnp.inf, work)
        wmax = jnp.max(vals, axis=-1, keepdims=True)
        ex = jnp.exp(vals - wmax)
        wts_ref[...] = ex / jnp.sum(ex, axis=-1, keepdims=True)
        sel_ref[...] = sel


@jax.jit
def _run(hidden_states, W1, b1, gamma, beta, W2, b2, routing_scores):
    features, sel, wts = pl.pallas_call(
        _fused_body,
        grid=(NSTEPS,),
        in_specs=[
            pl.BlockSpec((B, S, DCHUNK), lambda i: (0, 0, i)),
            pl.BlockSpec((ROUTER, DCHUNK), lambda i: (0, i)),
            pl.BlockSpec((1, ROUTER), lambda i: (0, 0)),
            pl.BlockSpec((1, ROUTER), lambda i: (0, 0)),
            pl.BlockSpec((1, ROUTER), lambda i: (0, 0)),
            pl.BlockSpec((ROUTER, ROUTER), lambda i: (0, 0)),
            pl.BlockSpec((1, ROUTER), lambda i: (0, 0)),
            pl.BlockSpec((1, NUM_EXPERTS), lambda i: (0, 0)),
        ],
        out_specs=[
            pl.BlockSpec((B, ROUTER), lambda i: (0, 0)),
            pl.BlockSpec((B, TOP_K), lambda i: (0, 0)),
            pl.BlockSpec((B, TOP_K), lambda i: (0, 0)),
        ],
        out_shape=[
            jax.ShapeDtypeStruct((B, ROUTER), jnp.float32),
            jax.ShapeDtypeStruct((B, TOP_K), jnp.int32),
            jax.ShapeDtypeStruct((B, TOP_K), jnp.float32),
        ],
        scratch_shapes=[pltpu.VMEM((B, ROUTER), jnp.float32)],
        compiler_params=pltpu.CompilerParams(
            dimension_semantics=("arbitrary",)),
    )(hidden_states, W1, b1.reshape(1, -1), gamma.reshape(1, -1),
      beta.reshape(1, -1), W2, b2.reshape(1, -1),
      routing_scores.reshape(1, -1))
    return features, sel[0], wts


def kernel(hidden_states, W1, b1, gamma, beta, W2, b2, routing_scores):
    return _run(hidden_states, W1, b1, gamma, beta, W2, b2, routing_scores)


# speedup vs baseline: 1.1392x; 1.0086x over previous
import jax
import jax.numpy as jnp
from jax.experimental import pallas as pl
from jax.experimental.pallas import tpu as pltpu

B, S, D_MODEL = 4, 2048, 4096
ROUTER = 1024
NUM_EXPERTS = 64
TOP_K = 8
THRESHOLD = 0.1
LATERAL = 0.1

DCHUNK = 1024
SCHUNK = 1024
NSTEPS = D_MODEL // DCHUNK
NSSTEPS = S // SCHUNK


def _fused_body(hid_ref, w1_ref, b1_ref, g_ref, be_ref, w2_ref, b2_ref,
                rs_ref, feat_ref, sel_ref, wts_ref, h_ref, pool_ref):
    i = pl.program_id(0)
    j = pl.program_id(1)

    part = jnp.sum(hid_ref[...], axis=1)

    @pl.when(j == 0)
    def _pinit():
        pool_ref[...] = part

    @pl.when(j > 0)
    def _pacc():
        pool_ref[...] += part

    @pl.when(j == NSSTEPS - 1)
    def _proj():
        pooled_j = pool_ref[...] * (1.0 / S)
        hj = jax.lax.dot_general(
            pooled_j, w1_ref[...], (((1,), (1,)), ((), ())),
            preferred_element_type=jnp.float32)

        @pl.when(i == 0)
        def _init():
            h_ref[...] = hj

        @pl.when(i > 0)
        def _acc():
            h_ref[...] += hj

    @pl.when((i == NSTEPS - 1) & (j == NSSTEPS - 1))
    def _head():
        h = h_ref[...] + b1_ref[...]
        mu = jnp.mean(h, axis=-1, keepdims=True)
        var = jnp.mean((h - mu) ** 2, axis=-1, keepdims=True)
        h = (h - mu) / jnp.sqrt(var + 1e-5) * g_ref[...] + be_ref[...]
        h = 0.5 * h * (1.0 + jax.lax.erf(h * (2.0 ** -0.5)))
        features = jax.lax.dot_general(
            h, w2_ref[...], (((1,), (1,)), ((), ())),
            preferred_element_type=jnp.float32) + b2_ref[...]
        feat_ref[...] = features

        affinity = jax.lax.dot_general(
            features, w2_ref[0:NUM_EXPERTS, :], (((1,), (1,)), ((), ())),
            preferred_element_type=jnp.float32)
        logits = rs_ref[...] + 0.1 * affinity

        acts = jnp.maximum(logits - THRESHOLD, 0.0)
        for _ in range(3):
            total = jnp.sum(acts, axis=-1, keepdims=True)
            inhibition = LATERAL * (total - acts)
            acts = jnp.maximum(logits - THRESHOLD - inhibition, 0.0)

        idx = jax.lax.broadcasted_iota(jnp.int32, (B, NUM_EXPERTS), 1)
        kidx = jax.lax.broadcasted_iota(jnp.int32, (B, TOP_K), 1)
        work = acts
        vals = jnp.zeros((B, TOP_K), jnp.float32)
        sel = jnp.zeros((B, TOP_K), jnp.int32)
        for k in range(TOP_K):
            m = jnp.max(work, axis=-1, keepdims=True)
            first = jnp.min(jnp.where(work == m, idx, NUM_EXPERTS),
                            axis=-1, keepdims=True)
            vals = jnp.where(kidx == k, m, vals)
            sel = jnp.where(kidx == k, first, sel)
            work = jnp.where(idx == first, -jnp.inf, work)
        wmax = jnp.max(vals, axis=-1, keepdims=True)
        ex = jnp.exp(vals - wmax)
        wts_ref[...] = ex / jnp.sum(ex, axis=-1, keepdims=True)
        sel_ref[...] = sel


@jax.jit
def _run(hidden_states, W1, b1, gamma, beta, W2, b2, routing_scores):
    features, sel, wts = pl.pallas_call(
        _fused_body,
        grid=(NSTEPS, NSSTEPS),
        in_specs=[
            pl.BlockSpec((B, SCHUNK, DCHUNK), lambda i, j: (0, j, i)),
            pl.BlockSpec((ROUTER, DCHUNK), lambda i, j: (0, i)),
            pl.BlockSpec((1, ROUTER), lambda i, j: (0, 0)),
            pl.BlockSpec((1, ROUTER), lambda i, j: (0, 0)),
            pl.BlockSpec((1, ROUTER), lambda i, j: (0, 0)),
            pl.BlockSpec((ROUTER, ROUTER), lambda i, j: (0, 0)),
            pl.BlockSpec((1, ROUTER), lambda i, j: (0, 0)),
            pl.BlockSpec((1, NUM_EXPERTS), lambda i, j: (0, 0)),
        ],
        out_specs=[
            pl.BlockSpec((B, ROUTER), lambda i, j: (0, 0)),
            pl.BlockSpec((B, TOP_K), lambda i, j: (0, 0)),
            pl.BlockSpec((B, TOP_K), lambda i, j: (0, 0)),
        ],
        out_shape=[
            jax.ShapeDtypeStruct((B, ROUTER), jnp.float32),
            jax.ShapeDtypeStruct((B, TOP_K), jnp.int32),
            jax.ShapeDtypeStruct((B, TOP_K), jnp.float32),
        ],
        scratch_shapes=[pltpu.VMEM((B, ROUTER), jnp.float32),
                        pltpu.VMEM((B, DCHUNK), jnp.float32)],
        compiler_params=pltpu.CompilerParams(
            dimension_semantics=("arbitrary", "arbitrary")),
    )(hidden_states, W1, b1.reshape(1, -1), gamma.reshape(1, -1),
      beta.reshape(1, -1), W2, b2.reshape(1, -1),
      routing_scores.reshape(1, -1))
    return features, sel[0], wts


def kernel(hidden_states, W1, b1, gamma, beta, W2, b2, routing_scores):
    return _run(hidden_states, W1, b1, gamma, beta, W2, b2, routing_scores)
